# Initial kernel scaffold; baseline (speedup 1.0000x reference)
import jax
import jax.numpy as jnp
from jax.experimental import pallas as pl


def kernel(logits_ulb_w, logits_ulb_s, tau_t, p_t, label_hist):
    raise NotImplementedError("write your pallas kernel here")


# speedup vs baseline: 1.2639x; 1.2639x over previous
import functools

import jax
import jax.numpy as jnp
from jax import lax
from jax.experimental import pallas as pl
from jax.experimental.pallas import tpu as pltpu
from jax.experimental.pallas import tpu_sc as plsc

SAT_EMA_K = 0.999
NROWS, NCLS = 16384, 1000
CPAD = 1024
BLK = 256
GRID = NROWS // BLK
NWORKERS = 32
CHUNK = NROWS // NWORKERS
LANES = 16


def _phase1_body(tau_ref, pt_ref, w_ref, s_ref,
                 mp_ref, idx_ref, nll_ref, thr_ref,
                 colsum_acc, mpsum_acc):
    i = pl.program_id(0)

    @pl.when(i == 0)
    def _init():
        colsum_acc[...] = jnp.zeros_like(colsum_acc)
        mpsum_acc[0] = 0.0

    w = w_ref[...]
    m = jnp.max(w, axis=1, keepdims=True)
    iota = lax.broadcasted_iota(jnp.int32, (BLK, NCLS), 1)
    idx = jnp.min(jnp.where(w == m, iota, NCLS), axis=1)
    ew = jnp.exp(w - m)
    sumexp = jnp.sum(ew, axis=1, keepdims=True)
    inv = 1.0 / sumexp
    mp = inv[:, 0]
    colsum_acc[:, :NCLS] += jnp.sum(ew * inv, axis=0, keepdims=True)
    mpsum_acc[0] += jnp.sum(mp)

    s = s_ref[...]
    ms = jnp.max(s, axis=1, keepdims=True)
    ses = jnp.sum(jnp.exp(s - ms), axis=1)
    lses = ms[:, 0] + jnp.log(ses)
    sval = jnp.max(jnp.where(iota == idx[:, None], s, -jnp.inf), axis=1)

    mp_ref[0, 0, :] = mp
    idx_ref[0, 0, :] = idx
    nll_ref[0, 0, :] = lses - sval

    @pl.when(i == GRID - 1)
    def _finish():
        p_new = pt_ref[...] * SAT_EMA_K + (1.0 - SAT_EMA_K) * (colsum_acc[...] / NROWS)
        tau_new = tau_ref[0] * SAT_EMA_K + (1.0 - SAT_EMA_K) * (mpsum_acc[0] / NROWS)
        thr_ref[...] = p_new * (tau_new / jnp.max(p_new))


def _phase1(w, s, tau, pt_pad):
    return pl.pallas_call(
        _phase1_body,
        grid=(GRID,),
        in_specs=[
            pl.BlockSpec(memory_space=pltpu.SMEM),
            pl.BlockSpec((1, CPAD), lambda i: (0, 0)),
            pl.BlockSpec((BLK, NCLS), lambda i: (i, 0)),
            pl.BlockSpec((BLK, NCLS), lambda i: (i, 0)),
        ],
        out_specs=[
            pl.BlockSpec((1, 1, BLK), lambda i: (i, 0, 0)),
            pl.BlockSpec((1, 1, BLK), lambda i: (i, 0, 0)),
            pl.BlockSpec((1, 1, BLK), lambda i: (i, 0, 0)),
            pl.BlockSpec((1, CPAD), lambda i: (0, 0)),
        ],
        out_shape=[
            jax.ShapeDtypeStruct((GRID, 1, BLK), jnp.float32),
            jax.ShapeDtypeStruct((GRID, 1, BLK), jnp.int32),
            jax.ShapeDtypeStruct((GRID, 1, BLK), jnp.float32),
            jax.ShapeDtypeStruct((1, CPAD), jnp.float32),
        ],
        scratch_shapes=[
            pltpu.VMEM((1, CPAD), jnp.float32),
            pltpu.SMEM((1,), jnp.float32),
        ],
    )(tau, pt_pad, w, s)


def _phase2_sc_body(idx_hbm, mp_hbm, nll_hbm, tbl_hbm,
                    mask_hbm, part_hbm,
                    idx_v, mp_v, nll_v, tbl_v, mask_v, acc_v):
    wid = lax.axis_index("s") * 2 + lax.axis_index("c")
    base = wid * CHUNK
    pltpu.sync_copy(idx_hbm.at[pl.ds(base, CHUNK)], idx_v)
    pltpu.sync_copy(mp_hbm.at[pl.ds(base, CHUNK)], mp_v)
    pltpu.sync_copy(nll_hbm.at[pl.ds(base, CHUNK)], nll_v)
    pltpu.sync_copy(tbl_hbm, tbl_v)

    def body(j, acc):
        o = j * LANES
        iv = idx_v[pl.ds(o, LANES)]
        thr = plsc.load_gather(tbl_v, [iv])
        mv = jnp.where(mp_v[pl.ds(o, LANES)] >= thr, 1.0, 0.0)
        mask_v[pl.ds(o, LANES)] = mv
        return acc + nll_v[pl.ds(o, LANES)] * mv

    acc = lax.fori_loop(0, CHUNK // LANES, body,
                        jnp.zeros((LANES,), jnp.float32))
    acc_v[...] = acc
    pltpu.sync_copy(mask_v, mask_hbm.at[pl.ds(base, CHUNK)])
    pltpu.sync_copy(acc_v, part_hbm.at[wid])


@functools.lru_cache(maxsize=1)
def _phase2():
    return pl.kernel(
        _phase2_sc_body,
        out_type=[
            jax.ShapeDtypeStruct((NROWS,), jnp.float32),
            jax.ShapeDtypeStruct((NWORKERS, LANES), jnp.float32),
        ],
        mesh=plsc.VectorSubcoreMesh(core_axis_name="c", subcore_axis_name="s"),
        compiler_params=pltpu.CompilerParams(needs_layout_passes=False),
        scratch_types=[
            pltpu.VMEM((CHUNK,), jnp.int32),
            pltpu.VMEM((CHUNK,), jnp.float32),
            pltpu.VMEM((CHUNK,), jnp.float32),
            pltpu.VMEM((CPAD,), jnp.float32),
            pltpu.VMEM((CHUNK,), jnp.float32),
            pltpu.VMEM((LANES,), jnp.float32),
        ],
    )


def kernel(logits_ulb_w, logits_ulb_s, tau_t, p_t, label_hist):
    del label_hist
    pt_pad = jnp.zeros((1, CPAD), jnp.float32).at[0, :NCLS].set(p_t)
    mp3, idx3, nll3, thr = _phase1(logits_ulb_w, logits_ulb_s,
                                   tau_t.reshape(1), pt_pad)
    mask, parts = _phase2()(idx3.reshape(NROWS), mp3.reshape(NROWS),
                            nll3.reshape(NROWS), thr.reshape(CPAD))
    loss = jnp.sum(parts) / NROWS
    return loss, mask


# speedup vs baseline: 1.4085x; 1.1144x over previous
import functools

import jax
import jax.numpy as jnp
from jax import lax
from jax.experimental import pallas as pl
from jax.experimental.pallas import tpu as pltpu
from jax.experimental.pallas import tpu_sc as plsc

SAT_EMA_K = 0.999
NROWS, NCLS = 16384, 1000
CPAD = 1024
BLK = 512
GRID = NROWS // BLK
NWORKERS = 32
CHUNK = NROWS // NWORKERS
LANES = 16


def _phase1_body(tau_ref, pt_ref, w_ref, s_ref,
                 mp_ref, idx_ref, nll_ref, thr_ref,
                 colsum_acc, mpsum_acc):
    i = pl.program_id(0)

    @pl.when(i == 0)
    def _init():
        colsum_acc[...] = jnp.zeros_like(colsum_acc)
        mpsum_acc[0] = 0.0

    w = w_ref[...]
    m = jnp.max(w, axis=1, keepdims=True)
    iota = lax.broadcasted_iota(jnp.int32, (BLK, NCLS), 1)
    idx = jnp.min(jnp.where(w == m, iota, NCLS), axis=1)
    ew = jnp.exp(w - m)
    sumexp = jnp.sum(ew, axis=1, keepdims=True)
    inv = 1.0 / sumexp
    mp = inv[:, 0]
    colsum_acc[:, :NCLS] += jnp.sum(ew * inv, axis=0, keepdims=True)
    mpsum_acc[0] += jnp.sum(mp)

    s = s_ref[...]
    ms = jnp.max(s, axis=1, keepdims=True)
    ses = jnp.sum(jnp.exp(s - ms), axis=1)
    lses = ms[:, 0] + jnp.log(ses)
    sval = jnp.max(jnp.where(iota == idx[:, None], s, -jnp.inf), axis=1)

    mp_ref[0, 0, :] = mp
    idx_ref[0, 0, :] = idx
    nll_ref[0, 0, :] = lses - sval

    @pl.when(i == GRID - 1)
    def _finish():
        p_new = pt_ref[...] * SAT_EMA_K + (1.0 - SAT_EMA_K) * (colsum_acc[...] / NROWS)
        tau_new = tau_ref[0] * SAT_EMA_K + (1.0 - SAT_EMA_K) * (mpsum_acc[0] / NROWS)
        thr_ref[...] = p_new * (tau_new / jnp.max(p_new))


def _phase1(w, s, tau, pt_pad):
    return pl.pallas_call(
        _phase1_body,
        grid=(GRID,),
        in_specs=[
            pl.BlockSpec(memory_space=pltpu.SMEM),
            pl.BlockSpec((1, CPAD), lambda i: (0, 0)),
            pl.BlockSpec((BLK, NCLS), lambda i: (i, 0)),
            pl.BlockSpec((BLK, NCLS), lambda i: (i, 0)),
        ],
        out_specs=[
            pl.BlockSpec((1, 1, BLK), lambda i: (i, 0, 0)),
            pl.BlockSpec((1, 1, BLK), lambda i: (i, 0, 0)),
            pl.BlockSpec((1, 1, BLK), lambda i: (i, 0, 0)),
            pl.BlockSpec((1, CPAD), lambda i: (0, 0)),
        ],
        out_shape=[
            jax.ShapeDtypeStruct((GRID, 1, BLK), jnp.float32),
            jax.ShapeDtypeStruct((GRID, 1, BLK), jnp.int32),
            jax.ShapeDtypeStruct((GRID, 1, BLK), jnp.float32),
            jax.ShapeDtypeStruct((1, CPAD), jnp.float32),
        ],
        scratch_shapes=[
            pltpu.VMEM((1, CPAD), jnp.float32),
            pltpu.SMEM((1,), jnp.float32),
        ],
    )(tau, pt_pad, w, s)


def _phase2_sc_body(idx_hbm, mp_hbm, nll_hbm, tbl_hbm,
                    mask_hbm, part_hbm,
                    idx_v, mp_v, nll_v, tbl_v, mask_v, acc_v):
    wid = lax.axis_index("s") * 2 + lax.axis_index("c")
    base = wid * CHUNK
    pltpu.sync_copy(idx_hbm.at[pl.ds(base, CHUNK)], idx_v)
    pltpu.sync_copy(mp_hbm.at[pl.ds(base, CHUNK)], mp_v)
    pltpu.sync_copy(nll_hbm.at[pl.ds(base, CHUNK)], nll_v)
    pltpu.sync_copy(tbl_hbm, tbl_v)

    def body(j, acc):
        o = j * LANES
        iv = idx_v[pl.ds(o, LANES)]
        thr = plsc.load_gather(tbl_v, [iv])
        mv = jnp.where(mp_v[pl.ds(o, LANES)] >= thr, 1.0, 0.0)
        mask_v[pl.ds(o, LANES)] = mv
        return acc + nll_v[pl.ds(o, LANES)] * mv

    acc = lax.fori_loop(0, CHUNK // LANES, body,
                        jnp.zeros((LANES,), jnp.float32))
    acc_v[...] = acc
    pltpu.sync_copy(mask_v, mask_hbm.at[pl.ds(base, CHUNK)])
    pltpu.sync_copy(acc_v, part_hbm.at[wid])


@functools.lru_cache(maxsize=1)
def _phase2():
    return pl.kernel(
        _phase2_sc_body,
        out_type=[
            jax.ShapeDtypeStruct((NROWS,), jnp.float32),
            jax.ShapeDtypeStruct((NWORKERS, LANES), jnp.float32),
        ],
        mesh=plsc.VectorSubcoreMesh(core_axis_name="c", subcore_axis_name="s"),
        compiler_params=pltpu.CompilerParams(needs_layout_passes=False),
        scratch_types=[
            pltpu.VMEM((CHUNK,), jnp.int32),
            pltpu.VMEM((CHUNK,), jnp.float32),
            pltpu.VMEM((CHUNK,), jnp.float32),
            pltpu.VMEM((CPAD,), jnp.float32),
            pltpu.VMEM((CHUNK,), jnp.float32),
            pltpu.VMEM((LANES,), jnp.float32),
        ],
    )


def kernel(logits_ulb_w, logits_ulb_s, tau_t, p_t, label_hist):
    del label_hist
    pt_pad = jnp.zeros((1, CPAD), jnp.float32).at[0, :NCLS].set(p_t)
    mp3, idx3, nll3, thr = _phase1(logits_ulb_w, logits_ulb_s,
                                   tau_t.reshape(1), pt_pad)
    mask, parts = _phase2()(idx3.reshape(NROWS), mp3.reshape(NROWS),
                            nll3.reshape(NROWS), thr.reshape(CPAD))
    loss = jnp.sum(parts) / NROWS
    return loss, mask
